# Initial kernel scaffold; baseline (speedup 1.0000x reference)
import jax
import jax.numpy as jnp
from jax.experimental import pallas as pl


def kernel(x, W, b):
    raise NotImplementedError("write your pallas kernel here")


# speedup vs baseline: 2.3562x; 2.3562x over previous
import jax
import jax.numpy as jnp
from jax.experimental import pallas as pl

_T, _P, _O, _C, _F, _K = 16, 64, 4, 32, 64, 128


def _embed_body(x_ref, w_ref, b_ref, out_ref):
    o = pl.program_id(1)
    xb = x_ref[0, :, 0, :, :].reshape(_P * _C, _F)
    acc = jax.lax.dot_general(
        xb, w_ref[o], (((1,), (0,)), ((), ())),
        preferred_element_type=jnp.float32)
    acc = acc + b_ref[o][None, :]
    acc = jnp.where(acc >= 0, acc, 0.1 * acc)
    out_ref[0] = acc.reshape(_P, _C, _K).transpose(1, 0, 2)


def kernel(x, W, b):
    outs = pl.pallas_call(
        _embed_body,
        grid=(_T, _O),
        in_specs=[
            pl.BlockSpec((1, _P, 1, _C, _F), lambda t, o: (t, 0, o, 0, 0)),
            pl.BlockSpec((_O, _F, _K), lambda t, o: (0, 0, 0)),
            pl.BlockSpec((_O, _K), lambda t, o: (0, 0)),
        ],
        out_specs=pl.BlockSpec((1, _C, _P, _K), lambda t, o: (t, o, 0, 0)),
        out_shape=jax.ShapeDtypeStruct((_T, _O * _C, _P, _K), jnp.float32),
    )(x, W, b)
    objCounts = jnp.full((_T, _P), _O * _C, dtype=jnp.int32)
    return outs, objCounts
